# baseline (speedup 1.0000x reference)
import jax
import jax.numpy as jnp
from jax import lax
from jax.experimental import pallas as pl
from jax.experimental.pallas import tpu as pltpu
from jax.experimental.pallas import tpu_sc as plsc

RESOS_K = (64, 128, 512)
COMBS_K = ((0, 1), (0, 2), (1, 2))
DK = 32
NPTS = 262144
NC = 2
NS = 16
NW = NC * NS
B = 32
PW = NPTS // NW
STEPS = PW // B
L = 16


def _sc_body(px, py, pz, t00, t01, t02, t10, t11, t12, t20, t21, t22,
             out_hbm, pts_v, idx0, idx1, idx2, wxy0, wxy1, wxy2,
             rows0, rows1, rows2, ov0, ov1, ov2,
             gsem0, gsem1, gsem2, osem0, osem1, osem2):
    tables = ((t00, t01, t02), (t10, t11, t12), (t20, t21, t22))
    idxs = (idx0, idx1, idx2)
    wxys = (wxy0, wxy1, wxy2)
    rows = (rows0, rows1, rows2)
    ovs = (ov0, ov1, ov2)
    gsems = (gsem0, gsem1, gsem2)
    osems = (osem0, osem1, osem2)
    wid = lax.axis_index("s") * NC + lax.axis_index("c")
    pbase = wid * PW

    pltpu.sync_copy(px.at[pl.ds(pbase, PW)], pts_v.at[0])
    pltpu.sync_copy(py.at[pl.ds(pbase, PW)], pts_v.at[1])
    pltpu.sync_copy(pz.at[pl.ds(pbase, PW)], pts_v.at[2])

    def emit_idx(s, cbase):
        R = RESOS_K[s]
        h = 0.5 * (R - 1)

        def idx_body(g, _):
            o = cbase + g * L
            oo = g * L
            c0, c1, w = [], [], []
            for d in range(3):
                t = (pts_v[d, pl.ds(o, L)] + 1.0) * h
                t = jnp.maximum(t, 0.0)
                f = jnp.minimum(t.astype(jnp.int32), R - 1)
                c0.append(f)
                c1.append(jnp.minimum(f + 1, R - 1))
                w.append(t - f.astype(jnp.float32))
            for p, (a, b) in enumerate(COMBS_K):
                idxs[s][p, 0, pl.ds(oo, L)] = c0[b] * R + c0[a]
                idxs[s][p, 1, pl.ds(oo, L)] = c0[b] * R + c1[a]
                idxs[s][p, 2, pl.ds(oo, L)] = c1[b] * R + c0[a]
                idxs[s][p, 3, pl.ds(oo, L)] = c1[b] * R + c1[a]
                w11 = w[a] * w[b]
                wxys[s][p, 0, pl.ds(oo, L)] = 1.0 - w[a] - w[b] + w11
                wxys[s][p, 1, pl.ds(oo, L)] = w[a] - w11
                wxys[s][p, 2, pl.ds(oo, L)] = w[b] - w11
                wxys[s][p, 3, pl.ds(oo, L)] = w11
            return 0

        lax.fori_loop(0, B // L, idx_body, 0)

    def fire_gathers(s):
        for p in range(3):
            for c in range(4):
                pltpu.async_copy(tables[s][p].at[idxs[s].at[p, c]],
                                 rows[s].at[p, c], gsems[s])

    def wait_gathers(s):
        for p in range(3):
            for c in range(4):
                pltpu.make_async_copy(tables[s][p].at[idxs[s].at[p, c]],
                                      rows[s].at[p, c], gsems[s]).wait()

    def emit_combine(s):
        def comb_body(g, _):
            o = g * L
            wv = [[wxys[s][p, q, pl.ds(o, L)] for q in range(4)]
                  for p in range(3)]
            for k in range(L):
                j = o + k
                lo = hi = None
                for p in range(3):
                    w00 = wv[p][0][k]
                    w01 = wv[p][1][k]
                    w10 = wv[p][2][k]
                    w11 = wv[p][3][k]
                    slo = (w00 * rows[s][p, 0, j, pl.ds(0, L)]
                           + w01 * rows[s][p, 1, j, pl.ds(0, L)]
                           + w10 * rows[s][p, 2, j, pl.ds(0, L)]
                           + w11 * rows[s][p, 3, j, pl.ds(0, L)])
                    shi = (w00 * rows[s][p, 0, j, pl.ds(L, L)]
                           + w01 * rows[s][p, 1, j, pl.ds(L, L)]
                           + w10 * rows[s][p, 2, j, pl.ds(L, L)]
                           + w11 * rows[s][p, 3, j, pl.ds(L, L)])
                    lo = slo if lo is None else lo * slo
                    hi = shi if hi is None else hi * shi
                ovs[s][j, pl.ds(0, L)] = lo
                ovs[s][j, pl.ds(L, L)] = hi
            return 0

        lax.fori_loop(0, B // L, comb_body, 0)

    for s in range(3):
        emit_idx(s, 0)
        fire_gathers(s)

    def stage_body(i, _):
        gbase = pbase + i * B
        for s in range(3):
            wait_gathers(s)

            @pl.when(i > 0)
            def _(s=s):
                pltpu.make_async_copy(
                    ovs[s],
                    out_hbm.at[pl.ds(pbase, B), pl.ds(DK * s, DK)],
                    osems[s]).wait()

            emit_combine(s)
            pltpu.async_copy(
                ovs[s], out_hbm.at[pl.ds(gbase, B), pl.ds(DK * s, DK)],
                osems[s])

            @pl.when(i < STEPS - 1)
            def _(s=s, i=i):
                emit_idx(s, (i + 1) * B)
                fire_gathers(s)

        return 0

    lax.fori_loop(0, STEPS, stage_body, 0)
    for s in range(3):
        pltpu.make_async_copy(
            ovs[s], out_hbm.at[pl.ds(pbase, B), pl.ds(DK * s, DK)],
            osems[s]).wait()


@jax.jit
def kernel(pts, aabb, g0_0, g0_1, g0_2, g1_0, g1_1, g1_2, g2_0, g2_1, g2_2):
    grids = ((g0_0, g0_1, g0_2), (g1_0, g1_1, g1_2), (g2_0, g2_1, g2_2))
    sc_ = 2.0 / (aabb[1] - aabb[0])
    cols = []
    for d in range(3):
        cols.append((pts[:, d] - aabb[0, d]) * sc_[d] - 1.0)
    px, py, pz = cols
    tabs = []
    for s, R in enumerate(RESOS_K):
        for p in range(3):
            tabs.append(jnp.transpose(grids[s][p][0], (1, 2, 0))
                        .reshape(R * R, DK))
    mesh = plsc.VectorSubcoreMesh(core_axis_name="c", subcore_axis_name="s")
    run = pl.kernel(
        _sc_body,
        mesh=mesh,
        compiler_params=pltpu.CompilerParams(use_tc_tiling_on_sc=False),
        out_type=jax.ShapeDtypeStruct((NPTS, 4 * DK), jnp.float32),
        scratch_types=(
            [pltpu.VMEM((3, PW), jnp.float32)]
            + [pltpu.VMEM((3, 4, B), jnp.int32)] * 3
            + [pltpu.VMEM((3, 4, B), jnp.float32)] * 3
            + [pltpu.VMEM((3, 4, B, DK), jnp.float32)] * 3
            + [pltpu.VMEM((B, DK), jnp.float32)] * 3
            + [pltpu.SemaphoreType.DMA] * 6
        ),
    )
    return run(px, py, pz, *tabs)[:, : 3 * DK]


# speedup vs baseline: 1.1866x; 1.1866x over previous
import jax
import jax.numpy as jnp
from jax import lax
from jax.experimental import pallas as pl
from jax.experimental.pallas import tpu as pltpu
from jax.experimental.pallas import tpu_sc as plsc

RESOS_K = (64, 128, 512)
COMBS_K = ((0, 1), (0, 2), (1, 2))
DK = 32
NPTS = 262144
NC = 2
NS = 16
NW = NC * NS
B = 64
PW = NPTS // NW
STEPS = PW // B
L = 16


def _sc_body(px, py, pz, t00, t01, t02, t10, t11, t12, t20, t21, t22,
             out_hbm, pts_v, idx0, idx1, idx2, wxy0, wxy1, wxy2,
             rows0, rows1, rows2, ov0, ov1, ov2,
             gsem0, gsem1, gsem2, osem0, osem1, osem2):
    tables = ((t00, t01, t02), (t10, t11, t12), (t20, t21, t22))
    idxs = (idx0, idx1, idx2)
    wxys = (wxy0, wxy1, wxy2)
    rows = (rows0, rows1, rows2)
    ovs = (ov0, ov1, ov2)
    gsems = (gsem0, gsem1, gsem2)
    osems = (osem0, osem1, osem2)
    wid = lax.axis_index("s") * NC + lax.axis_index("c")
    pbase = wid * PW

    pltpu.sync_copy(px.at[pl.ds(pbase, PW)], pts_v.at[0])
    pltpu.sync_copy(py.at[pl.ds(pbase, PW)], pts_v.at[1])
    pltpu.sync_copy(pz.at[pl.ds(pbase, PW)], pts_v.at[2])

    def emit_idx(s, cbase):
        R = RESOS_K[s]
        h = 0.5 * (R - 1)

        def idx_body(g, _):
            o = cbase + g * L
            oo = g * L
            c0, c1, w = [], [], []
            for d in range(3):
                t = (pts_v[d, pl.ds(o, L)] + 1.0) * h
                t = jnp.maximum(t, 0.0)
                f = jnp.minimum(t.astype(jnp.int32), R - 1)
                c0.append(f)
                c1.append(jnp.minimum(f + 1, R - 1))
                w.append(t - f.astype(jnp.float32))
            for p, (a, b) in enumerate(COMBS_K):
                idxs[s][p, 0, pl.ds(oo, L)] = c0[b] * R + c0[a]
                idxs[s][p, 1, pl.ds(oo, L)] = c0[b] * R + c1[a]
                idxs[s][p, 2, pl.ds(oo, L)] = c1[b] * R + c0[a]
                idxs[s][p, 3, pl.ds(oo, L)] = c1[b] * R + c1[a]
                w11 = w[a] * w[b]
                wxys[s][p, 0, pl.ds(oo, L)] = 1.0 - w[a] - w[b] + w11
                wxys[s][p, 1, pl.ds(oo, L)] = w[a] - w11
                wxys[s][p, 2, pl.ds(oo, L)] = w[b] - w11
                wxys[s][p, 3, pl.ds(oo, L)] = w11
            return 0

        lax.fori_loop(0, B // L, idx_body, 0)

    def fire_gathers(s):
        for p in range(3):
            for c in range(4):
                pltpu.async_copy(tables[s][p].at[idxs[s].at[p, c]],
                                 rows[s].at[p, c], gsems[s])

    def wait_gathers(s):
        for p in range(3):
            for c in range(4):
                pltpu.make_async_copy(tables[s][p].at[idxs[s].at[p, c]],
                                      rows[s].at[p, c], gsems[s]).wait()

    def emit_combine(s):
        def comb_body(g, _):
            o = g * L
            wv = [[wxys[s][p, q, pl.ds(o, L)] for q in range(4)]
                  for p in range(3)]
            for k in range(L):
                j = o + k
                lo = hi = None
                for p in range(3):
                    w00 = wv[p][0][k]
                    w01 = wv[p][1][k]
                    w10 = wv[p][2][k]
                    w11 = wv[p][3][k]
                    slo = (w00 * rows[s][p, 0, j, pl.ds(0, L)]
                           + w01 * rows[s][p, 1, j, pl.ds(0, L)]
                           + w10 * rows[s][p, 2, j, pl.ds(0, L)]
                           + w11 * rows[s][p, 3, j, pl.ds(0, L)])
                    shi = (w00 * rows[s][p, 0, j, pl.ds(L, L)]
                           + w01 * rows[s][p, 1, j, pl.ds(L, L)]
                           + w10 * rows[s][p, 2, j, pl.ds(L, L)]
                           + w11 * rows[s][p, 3, j, pl.ds(L, L)])
                    lo = slo if lo is None else lo * slo
                    hi = shi if hi is None else hi * shi
                ovs[s][j, pl.ds(0, L)] = lo
                ovs[s][j, pl.ds(L, L)] = hi
            return 0

        lax.fori_loop(0, B // L, comb_body, 0)

    for s in range(3):
        emit_idx(s, 0)
        fire_gathers(s)

    def stage_body(i, _):
        gbase = pbase + i * B
        for s in range(3):
            wait_gathers(s)

            @pl.when(i > 0)
            def _(s=s):
                pltpu.make_async_copy(
                    ovs[s],
                    out_hbm.at[pl.ds(pbase, B), pl.ds(DK * s, DK)],
                    osems[s]).wait()

            emit_combine(s)
            pltpu.async_copy(
                ovs[s], out_hbm.at[pl.ds(gbase, B), pl.ds(DK * s, DK)],
                osems[s])

            @pl.when(i < STEPS - 1)
            def _(s=s, i=i):
                emit_idx(s, (i + 1) * B)
                fire_gathers(s)

        return 0

    lax.fori_loop(0, STEPS, stage_body, 0)
    for s in range(3):
        pltpu.make_async_copy(
            ovs[s], out_hbm.at[pl.ds(pbase, B), pl.ds(DK * s, DK)],
            osems[s]).wait()


@jax.jit
def kernel(pts, aabb, g0_0, g0_1, g0_2, g1_0, g1_1, g1_2, g2_0, g2_1, g2_2):
    grids = ((g0_0, g0_1, g0_2), (g1_0, g1_1, g1_2), (g2_0, g2_1, g2_2))
    sc_ = 2.0 / (aabb[1] - aabb[0])
    cols = []
    for d in range(3):
        cols.append((pts[:, d] - aabb[0, d]) * sc_[d] - 1.0)
    px, py, pz = cols
    tabs = []
    for s, R in enumerate(RESOS_K):
        for p in range(3):
            tabs.append(jnp.transpose(grids[s][p][0], (1, 2, 0))
                        .reshape(R * R, DK))
    mesh = plsc.VectorSubcoreMesh(core_axis_name="c", subcore_axis_name="s")
    run = pl.kernel(
        _sc_body,
        mesh=mesh,
        compiler_params=pltpu.CompilerParams(use_tc_tiling_on_sc=False),
        out_type=jax.ShapeDtypeStruct((NPTS, 4 * DK), jnp.float32),
        scratch_types=(
            [pltpu.VMEM((3, PW), jnp.float32)]
            + [pltpu.VMEM((3, 4, B), jnp.int32)] * 3
            + [pltpu.VMEM((3, 4, B), jnp.float32)] * 3
            + [pltpu.VMEM((3, 4, B, DK), jnp.float32)] * 3
            + [pltpu.VMEM((B, DK), jnp.float32)] * 3
            + [pltpu.SemaphoreType.DMA] * 6
        ),
    )
    return run(px, py, pz, *tabs)[:, : 3 * DK]
